# baseline (speedup 1.0000x reference)
import functools

import jax
import jax.numpy as jnp
from jax import lax
from jax.experimental import pallas as pl
from jax.experimental.pallas import tpu as pltpu
from jax.experimental.pallas import tpu_sc as plsc

NC = 2
NS = 16
L = 16
NW = NC * NS
SLAB = 32
BAND = 8


def _lse_body(tab_ref, out_ref, tt_ref):
    t = tab_ref[...]
    m = jnp.max(t, axis=-1, keepdims=True)
    s = jnp.sum(jnp.exp(t - m), axis=-1, keepdims=True)
    out_ref[...] = m + jnp.log(s)
    tt_ref[...] = t.T


def _final_body(part_ref, out_ref):
    out_ref[...] = jnp.sum(part_ref[...]).reshape(1, 1)


def _make_sc_gather(T, V, B):
    nbands = SLAB // BAND
    mesh = plsc.VectorSubcoreMesh(core_axis_name="c", subcore_axis_name="s")

    @functools.partial(
        pl.kernel,
        out_type=(
            jax.ShapeDtypeStruct((T, V, B), jnp.float32),
            jax.ShapeDtypeStruct((NW, L), jnp.float32),
        ),
        mesh=mesh,
        compiler_params=pltpu.CompilerParams(
            use_tc_tiling_on_sc=True, needs_layout_passes=False
        ),
        scratch_types=[
            pltpu.VMEM((SLAB * 1024,), jnp.float32),
            pltpu.VMEM((BAND, V), jnp.float32),
            pltpu.VMEM((B,), jnp.int32),
            pltpu.VMEM((B,), jnp.int32),
            pltpu.VMEM((B,), jnp.int32),
            pltpu.VMEM((B,), jnp.int32),
            pltpu.VMEM((V,), jnp.float32),
            pltpu.VMEM((BAND, B), jnp.float32),
            pltpu.VMEM((BAND, B), jnp.float32),
            pltpu.VMEM((BAND, B), jnp.float32),
            pltpu.VMEM((BAND, B), jnp.float32),
            pltpu.VMEM((BAND, B), jnp.float32),
            pltpu.VMEM((BAND, B), jnp.float32),
            pltpu.VMEM((BAND, B), jnp.float32),
            pltpu.VMEM((BAND, B), jnp.float32),
            pltpu.VMEM((L,), jnp.float32),
            pltpu.SemaphoreType.DMA,
            pltpu.SemaphoreType.DMA,
            pltpu.SemaphoreType.DMA,
            pltpu.SemaphoreType.DMA,
        ],
    )
    def sc_gather(xt_hbm, yt_hbm, lse_hbm, tt_hbm,
                  p_hbm, part_hbm,
                  slab, stmp, xt0, xt1, yt0, yt1, lsev,
                  bb00, bb01, bb10, bb11, bb20, bb21, bb30, bb31,
                  accv, isem0, isem1, osem0, osem1):
        wid = lax.axis_index("s") * NC + lax.axis_index("c")
        c0 = SLAB * wid
        c1 = jnp.minimum(c0 + SLAB, V)
        slab_start = jnp.minimum(c0, V - SLAB)

        for s in range(SLAB // BAND):
            pltpu.sync_copy(
                tt_hbm.at[pl.ds(slab_start + BAND * s, BAND), :], stmp
            )

            def unpack_body(c, carry, s=s):
                cb = jnp.minimum(L * c, V - L)
                for j in range(BAND):
                    slab[pl.ds((BAND * s + j) * 1024 + cb, L)] = (
                        stmp[j, pl.ds(cb, L)]
                    )
                return carry

            lax.fori_loop(0, (V + L - 1) // L, unpack_body, 0)
        pltpu.sync_copy(lse_hbm, lsev)
        accv[...] = jnp.zeros((L,), jnp.float32)

        rowbase = []
        for k in range(nbands):
            for r in range(BAND):
                rowbase.append(
                    jnp.minimum(c0 + BAND * k - slab_start + r, SLAB - 1) * 1024
                )

        xt = (xt0, xt1)
        yt = (yt0, yt1)
        bb = ((bb00, bb01), (bb10, bb11), (bb20, bb21), (bb30, bb31))
        isem = (isem0, isem1)
        osem = (osem0, osem1)

        pltpu.async_copy(xt_hbm.at[0], xt0, isem0)
        pltpu.async_copy(yt_hbm.at[0], yt0, isem0)

        def do_t(t, p):
            @pl.when(t >= 2)
            def _():
                for k in range(nbands):
                    @pl.when(c0 + BAND * k < c1)
                    def _():
                        pltpu.make_async_copy(
                            bb[k][p], p_hbm.at[0, pl.ds(0, BAND), :], osem[p]
                        ).wait()

            pltpu.make_async_copy(xt_hbm.at[0], xt[p], isem[p]).wait()
            pltpu.make_async_copy(yt_hbm.at[0], yt[p], isem[p]).wait()

            @pl.when(t + 1 < T)
            def _():
                pltpu.async_copy(xt_hbm.at[t + 1], xt[1 - p], isem[1 - p])
                pltpu.async_copy(yt_hbm.at[t + 1], yt[1 - p], isem[1 - p])

            @plsc.parallel_loop(0, B // L, unroll=2)
            def _(g):
                gb = L * g
                xg = xt[p][pl.ds(gb, L)]
                vals = [
                    plsc.load_gather(slab, [xg + rowbase[i]])
                    for i in range(2 * BAND)
                ]
                for k in range(2):
                    for r in range(BAND):
                        bb[k][p][r, pl.ds(gb, L)] = vals[k * BAND + r]

            @plsc.parallel_loop(
                0, B // L, carry=jnp.zeros((L,), jnp.float32)
            )
            def g_acc(g, acc):
                gb = L * g
                xg = xt[p][pl.ds(gb, L)]
                yg = yt[p][pl.ds(gb, L)]
                vals = [
                    plsc.load_gather(slab, [xg + rowbase[2 * BAND + i]])
                    for i in range(2 * BAND)
                ]
                own = (yg >= c0) & (yg < c1)
                localc = jnp.clip(yg - slab_start, 0, SLAB - 1)
                tv = plsc.load_gather(slab, [xg + localc * 1024])
                for k in range(2):
                    for r in range(BAND):
                        bb[2 + k][p][r, pl.ds(gb, L)] = vals[k * BAND + r]
                return acc + jnp.where(own, tv, jnp.float32(0.0))

            accv[...] = accv[...] + g_acc

            for j in range(2):
                xg = xt[p][pl.ds(SLAB * wid + L * j, L)]
                accv[...] = accv[...] + plsc.load_gather(lsev, [xg])

            for k in range(nbands):
                @pl.when(c0 + BAND * k < c1)
                def _():
                    pltpu.async_copy(
                        bb[k][p],
                        p_hbm.at[t, pl.ds(c0 + BAND * k, BAND), :],
                        osem[p],
                    )

        def pair_body(i, carry):
            do_t(2 * i, 0)
            do_t(2 * i + 1, 1)
            return carry

        lax.fori_loop(0, T // 2, pair_body, 0)

        for p in range(2):
            for k in range(nbands):
                @pl.when(c0 + BAND * k < c1)
                def _():
                    pltpu.make_async_copy(
                        bb[k][p], p_hbm.at[0, pl.ds(0, BAND), :], osem[p]
                    ).wait()

        pltpu.sync_copy(accv, part_hbm.at[wid])

    return sc_gather


def kernel(x, y, table):
    B, T = x.shape
    V = table.shape[0]
    N = B * T
    xt = x.T.astype(jnp.int32)
    yt = y.T.astype(jnp.int32)

    lse2, tt = pl.pallas_call(
        _lse_body,
        out_shape=(
            jax.ShapeDtypeStruct((V, 1), jnp.float32),
            jax.ShapeDtypeStruct((V, V), jnp.float32),
        ),
    )(table)
    lse = lse2.reshape(V)

    p, parts = _make_sc_gather(T, V, B)(xt, yt, lse, tt)
    logits = jnp.transpose(p, (2, 0, 1))

    loss_sum = pl.pallas_call(
        _final_body,
        out_shape=jax.ShapeDtypeStruct((1, 1), jnp.float32),
    )(parts)
    loss = loss_sum[0, 0] / N
    return logits, loss


# speedup vs baseline: 1.0417x; 1.0417x over previous
import functools

import jax
import jax.numpy as jnp
from jax import lax
from jax.experimental import pallas as pl
from jax.experimental.pallas import tpu as pltpu
from jax.experimental.pallas import tpu_sc as plsc

NC = 2
NS = 16
L = 16
NW = NC * NS
SLAB = 32
BAND = 8


def _lse_body(tab_ref, out_ref, tt_ref):
    t = tab_ref[...]
    m = jnp.max(t, axis=-1, keepdims=True)
    s = jnp.sum(jnp.exp(t - m), axis=-1, keepdims=True)
    out_ref[...] = m + jnp.log(s)
    tt_ref[...] = t.T


def _final_body(part_ref, out_ref):
    out_ref[...] = jnp.sum(part_ref[...]).reshape(1, 1)


def _make_sc_gather(T, V, B):
    nbands = SLAB // BAND
    mesh = plsc.VectorSubcoreMesh(core_axis_name="c", subcore_axis_name="s")

    @functools.partial(
        pl.kernel,
        out_type=(
            jax.ShapeDtypeStruct((T, V, B), jnp.float32),
            jax.ShapeDtypeStruct((NW, L), jnp.float32),
        ),
        mesh=mesh,
        compiler_params=pltpu.CompilerParams(
            use_tc_tiling_on_sc=True, needs_layout_passes=False
        ),
        scratch_types=[
            pltpu.VMEM((SLAB * 1024,), jnp.float32),
            pltpu.VMEM((BAND, V), jnp.float32),
            pltpu.VMEM((B,), jnp.int32),
            pltpu.VMEM((B,), jnp.int32),
            pltpu.VMEM((B,), jnp.int32),
            pltpu.VMEM((B,), jnp.int32),
            pltpu.VMEM((V,), jnp.float32),
            pltpu.VMEM((BAND, B), jnp.float32),
            pltpu.VMEM((BAND, B), jnp.float32),
            pltpu.VMEM((BAND, B), jnp.float32),
            pltpu.VMEM((BAND, B), jnp.float32),
            pltpu.VMEM((BAND, B), jnp.float32),
            pltpu.VMEM((BAND, B), jnp.float32),
            pltpu.VMEM((BAND, B), jnp.float32),
            pltpu.VMEM((BAND, B), jnp.float32),
            pltpu.VMEM((L,), jnp.float32),
            pltpu.SemaphoreType.DMA,
            pltpu.SemaphoreType.DMA,
            pltpu.SemaphoreType.DMA,
            pltpu.SemaphoreType.DMA,
        ],
    )
    def sc_gather(xt_hbm, yt_hbm, lse_hbm, tt_hbm,
                  p_hbm, part_hbm,
                  slab, stmp, xt0, xt1, yt0, yt1, lsev,
                  bb00, bb01, bb10, bb11, bb20, bb21, bb30, bb31,
                  accv, isem0, isem1, osem0, osem1):
        wid = lax.axis_index("s") * NC + lax.axis_index("c")
        c0 = SLAB * wid
        c1 = jnp.minimum(c0 + SLAB, V)
        slab_start = jnp.minimum(c0, V - SLAB)

        for s in range(SLAB // BAND):
            pltpu.sync_copy(
                tt_hbm.at[pl.ds(slab_start + BAND * s, BAND), :], stmp
            )

            def unpack_body(c, carry, s=s):
                cb = jnp.minimum(L * c, V - L)
                for j in range(BAND):
                    slab[pl.ds((BAND * s + j) * 1024 + cb, L)] = (
                        stmp[j, pl.ds(cb, L)]
                    )
                return carry

            lax.fori_loop(0, (V + L - 1) // L, unpack_body, 0)
        pltpu.sync_copy(lse_hbm, lsev)
        accv[...] = jnp.zeros((L,), jnp.float32)

        rowbase = []
        for k in range(nbands):
            for r in range(BAND):
                rowbase.append(
                    jnp.minimum(c0 + BAND * k - slab_start + r, SLAB - 1) * 1024
                )

        xt = (xt0, xt1)
        yt = (yt0, yt1)
        bb = ((bb00, bb01), (bb10, bb11), (bb20, bb21), (bb30, bb31))
        isem = (isem0, isem1)
        osem = (osem0, osem1)

        pltpu.async_copy(xt_hbm.at[0], xt0, isem0)
        pltpu.async_copy(yt_hbm.at[0], yt0, isem0)

        def do_t(t, p):
            @pl.when(t >= 2)
            def _():
                for k in range(nbands):
                    @pl.when(c0 + BAND * k < c1)
                    def _():
                        pltpu.make_async_copy(
                            bb[k][p], p_hbm.at[0, pl.ds(0, BAND), :], osem[p]
                        ).wait()

            pltpu.make_async_copy(xt_hbm.at[0], xt[p], isem[p]).wait()
            pltpu.make_async_copy(yt_hbm.at[0], yt[p], isem[p]).wait()

            @pl.when(t + 1 < T)
            def _():
                pltpu.async_copy(xt_hbm.at[t + 1], xt[1 - p], isem[1 - p])
                pltpu.async_copy(yt_hbm.at[t + 1], yt[1 - p], isem[1 - p])

            @plsc.parallel_loop(
                0, B // L, carry=jnp.zeros((L,), jnp.float32)
            )
            def g_acc(g, acc):
                gb = L * g
                xg = xt[p][pl.ds(gb, L)]
                yg = yt[p][pl.ds(gb, L)]
                vals = [
                    plsc.load_gather(slab, [xg + rowbase[i]])
                    for i in range(2 * BAND)
                ]
                own = (yg >= c0) & (yg < c1)
                localc = jnp.clip(yg - slab_start, 0, SLAB - 1)
                tv = plsc.load_gather(slab, [xg + localc * 1024])
                for k in range(2):
                    for r in range(BAND):
                        bb[k][p][r, pl.ds(gb, L)] = vals[k * BAND + r]
                return acc + jnp.where(own, tv, jnp.float32(0.0))

            accv[...] = accv[...] + g_acc

            @plsc.parallel_loop(0, B // L)
            def _(g):
                gb = L * g
                xg = xt[p][pl.ds(gb, L)]
                vals = [
                    plsc.load_gather(slab, [xg + rowbase[2 * BAND + i]])
                    for i in range(2 * BAND)
                ]
                for k in range(2):
                    for r in range(BAND):
                        bb[2 + k][p][r, pl.ds(gb, L)] = vals[k * BAND + r]

            for j in range(2):
                xg = xt[p][pl.ds(SLAB * wid + L * j, L)]
                accv[...] = accv[...] + plsc.load_gather(lsev, [xg])

            for k in range(nbands):
                @pl.when(c0 + BAND * k < c1)
                def _():
                    pltpu.async_copy(
                        bb[k][p],
                        p_hbm.at[t, pl.ds(c0 + BAND * k, BAND), :],
                        osem[p],
                    )

        def pair_body(i, carry):
            do_t(2 * i, 0)
            do_t(2 * i + 1, 1)
            return carry

        lax.fori_loop(0, T // 2, pair_body, 0)

        for p in range(2):
            for k in range(nbands):
                @pl.when(c0 + BAND * k < c1)
                def _():
                    pltpu.make_async_copy(
                        bb[k][p], p_hbm.at[0, pl.ds(0, BAND), :], osem[p]
                    ).wait()

        pltpu.sync_copy(accv, part_hbm.at[wid])

    return sc_gather


def kernel(x, y, table):
    B, T = x.shape
    V = table.shape[0]
    N = B * T
    xt = x.T.astype(jnp.int32)
    yt = y.T.astype(jnp.int32)

    lse2, tt = pl.pallas_call(
        _lse_body,
        out_shape=(
            jax.ShapeDtypeStruct((V, 1), jnp.float32),
            jax.ShapeDtypeStruct((V, V), jnp.float32),
        ),
    )(table)
    lse = lse2.reshape(V)

    p, parts = _make_sc_gather(T, V, B)(xt, yt, lse, tt)
    logits = jnp.transpose(p, (2, 0, 1))

    loss_sum = pl.pallas_call(
        _final_body,
        out_shape=jax.ShapeDtypeStruct((1, 1), jnp.float32),
    )(parts)
    loss = loss_sum[0, 0] / N
    return logits, loss
